# Initial kernel scaffold; baseline (speedup 1.0000x reference)
import jax
import jax.numpy as jnp
from jax.experimental import pallas as pl


def kernel(x, weight):
    raise NotImplementedError("write your pallas kernel here")


# speedup vs baseline: 3.3049x; 3.3049x over previous
import functools

import jax
import jax.numpy as jnp
from jax import lax
from jax.experimental import pallas as pl
from jax.experimental.pallas import tpu as pltpu
from jax.experimental.pallas import tpu_sc as plsc

NC = 2
NS = 16
NW = NC * NS
CHUNK = 128
NBUF = 5


def _embedding_lookup(idx, weight, B, D, n_chunks):
    b_per_w = n_chunks * CHUNK
    mesh = plsc.VectorSubcoreMesh(core_axis_name="c", subcore_axis_name="s")

    @functools.partial(
        pl.kernel,
        out_type=jax.ShapeDtypeStruct((B, D), jnp.float32),
        mesh=mesh,
        scratch_types=[
            pltpu.VMEM((n_chunks, CHUNK), jnp.int32),
            [pltpu.VMEM((CHUNK, D), jnp.float32) for _ in range(NBUF)],
            [pltpu.SemaphoreType.DMA for _ in range(NBUF)],
            [pltpu.SemaphoreType.DMA for _ in range(NBUF)],
        ],
    )
    def emb(table_hbm, idx_hbm, out_hbm, idx_v, bufs, gsem, wsem):
        wid = lax.axis_index("s") * NC + lax.axis_index("c")
        base = wid * b_per_w
        pltpu.sync_copy(idx_hbm.at[wid], idx_v)

        def start_gather(c, b):
            pltpu.async_copy(table_hbm.at[idx_v.at[c]], bufs[b], gsem[b])

        def start_writeback(c, b):
            pltpu.async_copy(
                bufs[b], out_hbm.at[pl.ds(base + c * CHUNK, CHUNK)], wsem[b]
            )

        for b in range(NBUF):
            start_gather(b, b)

        @pl.loop(0, n_chunks - NBUF, step=NBUF)
        def _(j):
            for b in range(NBUF):
                pltpu.make_async_copy(table_hbm.at[idx_v.at[j + b]],
                                      bufs[b], gsem[b]).wait()
                start_writeback(j + b, b)
            for b in range(NBUF):
                pltpu.make_async_copy(
                    bufs[b], out_hbm.at[pl.ds(base, CHUNK)], wsem[b]
                ).wait()
                start_gather(j + b + NBUF, b)

        last = n_chunks - NBUF
        for b in range(NBUF):
            pltpu.make_async_copy(table_hbm.at[idx_v.at[last + b]],
                                  bufs[b], gsem[b]).wait()
            start_writeback(last + b, b)
        for b in range(NBUF):
            pltpu.make_async_copy(
                bufs[b], out_hbm.at[pl.ds(base, CHUNK)], wsem[b]
            ).wait()

    return emb(weight, idx)


def kernel(x, weight):
    B0, B1 = x.shape
    B = B0 * B1
    V, D = weight.shape
    assert B % (NW * CHUNK * NBUF) == 0
    n_chunks = B // (NW * CHUNK)
    idx = x.reshape(NW, n_chunks, CHUNK).astype(jnp.int32)
    out = _embedding_lookup(idx, weight, B, D, n_chunks)
    return out.reshape(B0, B1, D)


# speedup vs baseline: 5.9130x; 1.7891x over previous
import functools

import jax
import jax.numpy as jnp
from jax import lax
from jax.experimental import pallas as pl
from jax.experimental.pallas import tpu as pltpu
from jax.experimental.pallas import tpu_sc as plsc

NC = 2
NS = 16
NW = NC * NS
RPC = 2
NBUF = 4


def _embedding_lookup(idx, weight, B0, B1, D, n_chunks):
    rows_per_w = B0 // NW
    cpr = RPC * B1
    mesh = plsc.VectorSubcoreMesh(core_axis_name="c", subcore_axis_name="s")

    @functools.partial(
        pl.kernel,
        out_type=jax.ShapeDtypeStruct((B0, B1, D), jnp.float32),
        mesh=mesh,
        scratch_types=[
            pltpu.VMEM((n_chunks, cpr), jnp.int32),
            [pltpu.VMEM((cpr, D), jnp.float32) for _ in range(NBUF)],
            [pltpu.SemaphoreType.DMA for _ in range(NBUF)],
            [pltpu.SemaphoreType.DMA for _ in range(NBUF)],
        ],
    )
    def emb(table_hbm, idx_hbm, out_hbm, idx_v, bufs, gsem, wsem):
        wid = lax.axis_index("s") * NC + lax.axis_index("c")
        row0 = wid * rows_per_w
        pltpu.sync_copy(idx_hbm.at[wid], idx_v)

        def start_gather(c, b):
            pltpu.async_copy(table_hbm.at[idx_v.at[c]], bufs[b], gsem[b])

        def wait_gather(c, b):
            pltpu.make_async_copy(table_hbm.at[idx_v.at[c]],
                                  bufs[b], gsem[b]).wait()

        def start_writeback(c, b):
            for r in range(RPC):
                pltpu.async_copy(bufs[b].at[pl.ds(r * B1, B1)],
                                 out_hbm.at[row0 + c * RPC + r], wsem[b])

        def wait_writeback(c, b):
            for r in range(RPC):
                pltpu.make_async_copy(bufs[b].at[pl.ds(r * B1, B1)],
                                      out_hbm.at[row0], wsem[b]).wait()

        for b in range(NBUF):
            start_gather(b, b)

        @pl.loop(0, n_chunks - NBUF, step=NBUF)
        def _(j):
            for b in range(NBUF):
                wait_gather(j + b, b)
                start_writeback(j + b, b)
            for b in range(NBUF):
                wait_writeback(j + b, b)
                start_gather(j + b + NBUF, b)

        last = n_chunks - NBUF
        for b in range(NBUF):
            wait_gather(last + b, b)
            start_writeback(last + b, b)
        for b in range(NBUF):
            wait_writeback(last + b, b)

    return emb(weight, idx)


def kernel(x, weight):
    B0, B1 = x.shape
    V, D = weight.shape
    n_chunks = B0 // (NW * RPC)
    assert B0 % (NW * RPC) == 0 and n_chunks % NBUF == 0
    idx = x.reshape(NW, n_chunks, RPC * B1).astype(jnp.int32)
    return _embedding_lookup(idx, weight, B0, B1, D, n_chunks)


# speedup vs baseline: 10.1629x; 1.7187x over previous
import functools

import jax
import jax.numpy as jnp
from jax import lax
from jax.experimental import pallas as pl
from jax.experimental.pallas import tpu as pltpu
from jax.experimental.pallas import tpu_sc as plsc

NC = 2
NS = 16
NW = NC * NS
CHUNK = 128
NBUF = 5


def _embedding_lookup(idx, weight, B, D, n_chunks):
    b_per_w = n_chunks * CHUNK
    mesh = plsc.VectorSubcoreMesh(core_axis_name="c", subcore_axis_name="s")

    @functools.partial(
        pl.kernel,
        out_type=jax.ShapeDtypeStruct((B, D), jnp.float32),
        mesh=mesh,
        scratch_types=[
            pltpu.VMEM((n_chunks, CHUNK), jnp.int32),
            [pltpu.VMEM((CHUNK, D), jnp.float32) for _ in range(NBUF)],
            [pltpu.SemaphoreType.DMA for _ in range(NBUF)],
            [pltpu.SemaphoreType.DMA for _ in range(NBUF)],
        ],
    )
    def emb(table_hbm, idx_hbm, out_hbm, idx_v, bufs, gsem, wsem):
        wid = lax.axis_index("s") * NC + lax.axis_index("c")
        base = wid * b_per_w
        pltpu.sync_copy(idx_hbm.at[wid], idx_v)

        def start_gather(c, b):
            pltpu.async_copy(table_hbm.at[idx_v.at[c]], bufs[b], gsem[b])

        def wait_gather(c, b):
            pltpu.make_async_copy(table_hbm.at[idx_v.at[c]],
                                  bufs[b], gsem[b]).wait()

        def start_writeback(c, b):
            pltpu.async_copy(
                bufs[b], out_hbm.at[pl.ds(base + c * CHUNK, CHUNK)], wsem[b]
            )

        def wait_writeback(b):
            pltpu.make_async_copy(
                bufs[b], out_hbm.at[pl.ds(base, CHUNK)], wsem[b]
            ).wait()

        for b in range(NBUF):
            start_gather(b, b)

        @pl.loop(0, n_chunks - NBUF, step=NBUF)
        def _(j):
            for b in range(NBUF):
                wait_gather(j + b, b)
                start_writeback(j + b, b)
            for b in range(NBUF):
                wait_writeback(b)
                start_gather(j + b + NBUF, b)

        last = n_chunks - NBUF
        for b in range(NBUF):
            wait_gather(last + b, b)
            start_writeback(last + b, b)
        for b in range(NBUF):
            wait_writeback(b)

    return emb(weight, idx)


def kernel(x, weight):
    B0, B1 = x.shape
    B = B0 * B1
    V, D = weight.shape
    assert B % (NW * CHUNK * NBUF) == 0
    n_chunks = B // (NW * CHUNK)
    idx = x.T.reshape(NW, n_chunks, CHUNK).astype(jnp.int32)
    out = _embedding_lookup(idx, weight, B, D, n_chunks)
    return out.reshape(B1, B0, D).transpose(1, 0, 2)


# speedup vs baseline: 10.2521x; 1.0088x over previous
import functools

import jax
import jax.numpy as jnp
from jax import lax
from jax.experimental import pallas as pl
from jax.experimental.pallas import tpu as pltpu
from jax.experimental.pallas import tpu_sc as plsc

NC = 2
NS = 16
NW = NC * NS
CHUNK = 64
NBUF = 10


def _embedding_lookup(idx, weight, B, D, n_chunks):
    b_per_w = n_chunks * CHUNK
    mesh = plsc.VectorSubcoreMesh(core_axis_name="c", subcore_axis_name="s")

    @functools.partial(
        pl.kernel,
        out_type=jax.ShapeDtypeStruct((B, D), jnp.float32),
        mesh=mesh,
        scratch_types=[
            pltpu.VMEM((n_chunks, CHUNK), jnp.int32),
            [pltpu.VMEM((CHUNK, D), jnp.float32) for _ in range(NBUF)],
            [pltpu.SemaphoreType.DMA for _ in range(NBUF)],
            [pltpu.SemaphoreType.DMA for _ in range(NBUF)],
        ],
    )
    def emb(table_hbm, idx_hbm, out_hbm, idx_v, bufs, gsem, wsem):
        wid = lax.axis_index("s") * NC + lax.axis_index("c")
        base = wid * b_per_w
        pltpu.sync_copy(idx_hbm.at[wid], idx_v)

        def start_gather(c, b):
            pltpu.async_copy(table_hbm.at[idx_v.at[c]], bufs[b], gsem[b])

        def wait_gather(c, b):
            pltpu.make_async_copy(table_hbm.at[idx_v.at[c]],
                                  bufs[b], gsem[b]).wait()

        def start_writeback(c, b):
            pltpu.async_copy(
                bufs[b], out_hbm.at[pl.ds(base + c * CHUNK, CHUNK)], wsem[b]
            )

        def wait_writeback(b):
            pltpu.make_async_copy(
                bufs[b], out_hbm.at[pl.ds(base, CHUNK)], wsem[b]
            ).wait()

        for b in range(NBUF):
            start_gather(b, b)

        @pl.loop(0, n_chunks - NBUF, step=NBUF)
        def _(j):
            for b in range(NBUF):
                wait_gather(j + b, b)
                start_writeback(j + b, b)
            for b in range(NBUF):
                wait_writeback(b)
                start_gather(j + b + NBUF, b)

        last = n_chunks - NBUF
        for b in range(NBUF):
            wait_gather(last + b, b)
            start_writeback(last + b, b)
        for b in range(NBUF):
            wait_writeback(b)

    return emb(weight, idx)


def kernel(x, weight):
    B0, B1 = x.shape
    B = B0 * B1
    V, D = weight.shape
    assert B % (NW * CHUNK * NBUF) == 0
    n_chunks = B // (NW * CHUNK)
    idx = x.T.reshape(NW, n_chunks, CHUNK).astype(jnp.int32)
    out = _embedding_lookup(idx, weight, B, D, n_chunks)
    return out.reshape(B1, B0, D).transpose(1, 0, 2)


# speedup vs baseline: 10.4022x; 1.0146x over previous
import functools

import jax
import jax.numpy as jnp
from jax import lax
from jax.experimental import pallas as pl
from jax.experimental.pallas import tpu as pltpu
from jax.experimental.pallas import tpu_sc as plsc

NC = 2
NS = 16
NW = NC * NS
NBUF = 5


def _embedding_lookup(idx_t, weight, B0, B1, D):
    chunk = B0 // NW
    n_chunks = B1
    mesh = plsc.VectorSubcoreMesh(core_axis_name="c", subcore_axis_name="s")

    @functools.partial(
        pl.kernel,
        out_type=jax.ShapeDtypeStruct((B0 * B1, D), jnp.float32),
        mesh=mesh,
        scratch_types=[
            pltpu.VMEM((n_chunks, chunk), jnp.int32),
            [pltpu.VMEM((chunk, D), jnp.float32) for _ in range(NBUF)],
            [pltpu.SemaphoreType.DMA for _ in range(NBUF)],
            [pltpu.SemaphoreType.DMA for _ in range(NBUF)],
        ],
    )
    def emb(table_hbm, idx_hbm, out_hbm, idx_v, bufs, gsem, wsem):
        wid = lax.axis_index("s") * NC + lax.axis_index("c")
        col0 = wid * chunk
        pltpu.sync_copy(idx_hbm.at[:, pl.ds(col0, chunk)], idx_v)

        def start_gather(c, b):
            pltpu.async_copy(table_hbm.at[idx_v.at[c]], bufs[b], gsem[b])

        def wait_gather(c, b):
            pltpu.make_async_copy(table_hbm.at[idx_v.at[c]],
                                  bufs[b], gsem[b]).wait()

        def start_writeback(c, b):
            pltpu.async_copy(
                bufs[b], out_hbm.at[pl.ds(c * B0 + col0, chunk)], wsem[b]
            )

        def wait_writeback(b):
            pltpu.make_async_copy(
                bufs[b], out_hbm.at[pl.ds(col0, chunk)], wsem[b]
            ).wait()

        for b in range(NBUF):
            start_gather(b, b)

        @pl.loop(0, n_chunks - NBUF, step=NBUF)
        def _(j):
            for b in range(NBUF):
                wait_gather(j + b, b)
                start_writeback(j + b, b)
            for b in range(NBUF):
                wait_writeback(b)
                start_gather(j + b + NBUF, b)

        last = n_chunks - NBUF
        for b in range(NBUF):
            wait_gather(last + b, b)
            start_writeback(last + b, b)
        for b in range(NBUF):
            wait_writeback(b)

    return emb(weight, idx_t)


def kernel(x, weight):
    B0, B1 = x.shape
    V, D = weight.shape
    assert B0 % NW == 0 and B1 % NBUF == 0 and B0 // NW <= 128
    idx_t = x.T.astype(jnp.int32)
    out = _embedding_lookup(idx_t, weight, B0, B1, D)
    return out.reshape(B1, B0, D).transpose(1, 0, 2)


# speedup vs baseline: 10.4677x; 1.0063x over previous
import functools

import jax
import jax.numpy as jnp
from jax import lax
from jax.experimental import pallas as pl
from jax.experimental.pallas import tpu as pltpu
from jax.experimental.pallas import tpu_sc as plsc

NC = 2
NS = 16
NW = NC * NS
NBUF = 5


def _embedding_lookup(idx_t, weight, B0, B1, D):
    chunk = B0 // NW
    n_chunks = B1
    mesh = plsc.VectorSubcoreMesh(core_axis_name="c", subcore_axis_name="s")

    @functools.partial(
        pl.kernel,
        out_type=jax.ShapeDtypeStruct((B0 * B1, D), jnp.float32),
        mesh=mesh,
        scratch_types=[
            pltpu.VMEM((n_chunks, chunk), jnp.int32),
            [pltpu.VMEM((chunk, D), jnp.float32) for _ in range(NBUF)],
            [pltpu.SemaphoreType.DMA for _ in range(NBUF)],
            [pltpu.SemaphoreType.DMA for _ in range(NBUF)],
        ],
    )
    def emb(table_hbm, idx_hbm, out_hbm, idx_v, bufs, gsem, wsem):
        wid = lax.axis_index("s") * NC + lax.axis_index("c")
        col0 = wid * chunk
        pltpu.sync_copy(idx_hbm.at[pl.ds(0, 8), pl.ds(col0, chunk)],
                        idx_v.at[pl.ds(0, 8)])

        def start_gather(c, b):
            pltpu.async_copy(table_hbm.at[idx_v.at[c]], bufs[b], gsem[b])

        def wait_gather(c, b):
            pltpu.make_async_copy(table_hbm.at[idx_v.at[c]],
                                  bufs[b], gsem[b]).wait()

        def start_writeback(c, b):
            pltpu.async_copy(
                bufs[b], out_hbm.at[pl.ds(c * B0 + col0, chunk)], wsem[b]
            )

        def wait_writeback(b):
            pltpu.make_async_copy(
                bufs[b], out_hbm.at[pl.ds(col0, chunk)], wsem[b]
            ).wait()

        for b in range(NBUF):
            start_gather(b, b)
        pltpu.sync_copy(
            idx_hbm.at[pl.ds(8, n_chunks - 8), pl.ds(col0, chunk)],
            idx_v.at[pl.ds(8, n_chunks - 8)])

        @pl.loop(0, n_chunks - NBUF, step=NBUF)
        def _(j):
            for b in range(NBUF):
                wait_gather(j + b, b)
                start_writeback(j + b, b)
            for b in range(NBUF):
                wait_writeback(b)
                start_gather(j + b + NBUF, b)

        last = n_chunks - NBUF
        for b in range(NBUF):
            wait_gather(last + b, b)
            start_writeback(last + b, b)
        for b in range(NBUF):
            wait_writeback(b)

    return emb(weight, idx_t)


def kernel(x, weight):
    B0, B1 = x.shape
    V, D = weight.shape
    assert B0 % NW == 0 and B1 % NBUF == 0 and B0 // NW <= 128
    idx_t = x.T.astype(jnp.int32)
    out = _embedding_lookup(idx_t, weight, B0, B1, D)
    return out.reshape(B1, B0, D).transpose(1, 0, 2)
